# Initial kernel scaffold; baseline (speedup 1.0000x reference)
import jax
import jax.numpy as jnp
from jax.experimental import pallas as pl


def kernel(x, enc_w1, enc_b1, enc_w2, enc_b2, init_nodes, gin1_w1, gin1_b1, gin1_w2, gin1_b2, gin2_w1, gin2_b1, gin2_w2, gin2_b2, dec_w, dec_b):
    raise NotImplementedError("write your pallas kernel here")


# speedup vs baseline: 1.7019x; 1.7019x over previous
import functools

import jax
import jax.numpy as jnp
from jax.experimental import pallas as pl

_BLK = 512


def _fused_mlp_kernel(x_ref, w1_ref, b1_ref, w2_ref, b2_ref,
                      w3_ref, b3_ref, w4_ref, b4_ref,
                      w5_ref, b5_ref, w6_ref, b6_ref,
                      wd_ref, bd_ref, out_ref):
    f32 = jnp.float32
    h = x_ref[...]
    h = jnp.maximum(jnp.dot(h, w1_ref[...], preferred_element_type=f32) + b1_ref[...], 0.0)
    h = jnp.maximum(jnp.dot(h, w2_ref[...], preferred_element_type=f32) + b2_ref[...], 0.0)
    h = jnp.maximum(jnp.dot(h, w3_ref[...], preferred_element_type=f32) + b3_ref[...], 0.0)
    h = jnp.maximum(jnp.dot(h, w4_ref[...], preferred_element_type=f32) + b4_ref[...], 0.0)
    h = jnp.maximum(jnp.dot(h, w5_ref[...], preferred_element_type=f32) + b5_ref[...], 0.0)
    h = jnp.maximum(jnp.dot(h, w6_ref[...], preferred_element_type=f32) + b6_ref[...], 0.0)
    y = jnp.dot(h, wd_ref[...], preferred_element_type=f32) + bd_ref[...]
    out_ref[...] = y


@functools.partial(jax.jit, static_argnames=())
def kernel(x, enc_w1, enc_b1, enc_w2, enc_b2, init_nodes,
           gin1_w1, gin1_b1, gin1_w2, gin1_b2,
           gin2_w1, gin2_b1, gin2_w2, gin2_b2,
           dec_w, dec_b):
    B, D = x.shape
    H = enc_w1.shape[0]

    c0 = init_nodes.sum(axis=0)
    w1 = enc_w1.T
    w2 = enc_w2.T
    w3 = 3.0 * gin1_w1.T
    b3 = gin1_b1 + c0 @ gin1_w1.T
    w4 = gin1_w2.T
    w5 = 3.0 * gin2_w1.T
    w6 = gin2_w2.T
    wd = jnp.tile(dec_w.T, (1, 3))
    bd = jnp.broadcast_to(dec_b, (3,))

    def as_row(v):
        return v.reshape(1, -1).astype(jnp.float32)

    biases = [as_row(enc_b1), as_row(enc_b2), as_row(b3), as_row(gin1_b2),
              as_row(gin2_b1), as_row(gin2_b2), as_row(bd)]

    full = lambda a: pl.BlockSpec(a.shape, lambda i: (0, 0))
    in_specs = [pl.BlockSpec((_BLK, D), lambda i: (i, 0))]
    operands = [x]
    for w, b in zip([w1, w2, w3, w4, w5, w6, wd], biases):
        operands += [w.astype(jnp.float32), b]
        in_specs += [full(w), full(b)]

    out = pl.pallas_call(
        _fused_mlp_kernel,
        grid=(B // _BLK,),
        in_specs=in_specs,
        out_specs=pl.BlockSpec((_BLK, 3), lambda i: (i, 0)),
        out_shape=jax.ShapeDtypeStruct((B, 3), jnp.float32),
    )(*operands)
    return out


# speedup vs baseline: 1.9496x; 1.1455x over previous
import functools

import jax
import jax.numpy as jnp
from jax.experimental import pallas as pl

_BLK = 1024


def _fused_mlp_kernel(x_ref, w1_ref, b1_ref, w2_ref, b2_ref,
                      w3_ref, b3_ref, w4_ref, b4_ref,
                      w5_ref, b5_ref, w6_ref, b6_ref,
                      wd_ref, bd_ref, out_ref):
    f32 = jnp.float32
    h = x_ref[...]
    h = jnp.maximum(jnp.dot(h, w1_ref[...], preferred_element_type=f32) + b1_ref[...], 0.0)
    h = jnp.maximum(jnp.dot(h, w2_ref[...], preferred_element_type=f32) + b2_ref[...], 0.0)
    h = jnp.maximum(jnp.dot(h, w3_ref[...], preferred_element_type=f32) + b3_ref[...], 0.0)
    h = jnp.maximum(jnp.dot(h, w4_ref[...], preferred_element_type=f32) + b4_ref[...], 0.0)
    h = jnp.maximum(jnp.dot(h, w5_ref[...], preferred_element_type=f32) + b5_ref[...], 0.0)
    h = jnp.maximum(jnp.dot(h, w6_ref[...], preferred_element_type=f32) + b6_ref[...], 0.0)
    y = jnp.dot(h, wd_ref[...], preferred_element_type=f32) + bd_ref[...]
    out_ref[...] = y


@functools.partial(jax.jit, static_argnames=())
def kernel(x, enc_w1, enc_b1, enc_w2, enc_b2, init_nodes,
           gin1_w1, gin1_b1, gin1_w2, gin1_b2,
           gin2_w1, gin2_b1, gin2_w2, gin2_b2,
           dec_w, dec_b):
    B, D = x.shape
    H = enc_w1.shape[0]

    c0 = init_nodes.sum(axis=0)
    w1 = enc_w1.T
    w2 = enc_w2.T
    w3 = 3.0 * gin1_w1.T
    b3 = gin1_b1 + c0 @ gin1_w1.T
    w4 = gin1_w2.T
    w5 = 3.0 * gin2_w1.T
    w6 = gin2_w2.T
    wd = jnp.tile(dec_w.T, (1, 3))
    bd = jnp.broadcast_to(dec_b, (3,))

    def as_row(v):
        return v.reshape(1, -1).astype(jnp.float32)

    biases = [as_row(enc_b1), as_row(enc_b2), as_row(b3), as_row(gin1_b2),
              as_row(gin2_b1), as_row(gin2_b2), as_row(bd)]

    full = lambda a: pl.BlockSpec(a.shape, lambda i: (0, 0))
    in_specs = [pl.BlockSpec((_BLK, D), lambda i: (i, 0))]
    operands = [x]
    for w, b in zip([w1, w2, w3, w4, w5, w6, wd], biases):
        operands += [w.astype(jnp.float32), b]
        in_specs += [full(w), full(b)]

    out = pl.pallas_call(
        _fused_mlp_kernel,
        grid=(B // _BLK,),
        in_specs=in_specs,
        out_specs=pl.BlockSpec((_BLK, 3), lambda i: (i, 0)),
        out_shape=jax.ShapeDtypeStruct((B, 3), jnp.float32),
    )(*operands)
    return out


# speedup vs baseline: 2.0706x; 1.0621x over previous
import functools

import jax
import jax.numpy as jnp
from jax.experimental import pallas as pl

_BLK = 2048


def _fused_mlp_kernel(x_ref, w1_ref, b1_ref, w2_ref, b2_ref,
                      w3_ref, b3_ref, w4_ref, b4_ref,
                      w5_ref, b5_ref, w6_ref, b6_ref,
                      wd_ref, bd_ref, out_ref):
    f32 = jnp.float32
    h = x_ref[...]
    h = jnp.maximum(jnp.dot(h, w1_ref[...], preferred_element_type=f32) + b1_ref[...], 0.0)
    h = jnp.maximum(jnp.dot(h, w2_ref[...], preferred_element_type=f32) + b2_ref[...], 0.0)
    h = jnp.maximum(jnp.dot(h, w3_ref[...], preferred_element_type=f32) + b3_ref[...], 0.0)
    h = jnp.maximum(jnp.dot(h, w4_ref[...], preferred_element_type=f32) + b4_ref[...], 0.0)
    h = jnp.maximum(jnp.dot(h, w5_ref[...], preferred_element_type=f32) + b5_ref[...], 0.0)
    h = jnp.maximum(jnp.dot(h, w6_ref[...], preferred_element_type=f32) + b6_ref[...], 0.0)
    y = jnp.dot(h, wd_ref[...], preferred_element_type=f32) + bd_ref[...]
    out_ref[...] = y


@functools.partial(jax.jit, static_argnames=())
def kernel(x, enc_w1, enc_b1, enc_w2, enc_b2, init_nodes,
           gin1_w1, gin1_b1, gin1_w2, gin1_b2,
           gin2_w1, gin2_b1, gin2_w2, gin2_b2,
           dec_w, dec_b):
    B, D = x.shape
    H = enc_w1.shape[0]

    c0 = init_nodes.sum(axis=0)
    w1 = enc_w1.T
    w2 = enc_w2.T
    w3 = 3.0 * gin1_w1.T
    b3 = gin1_b1 + c0 @ gin1_w1.T
    w4 = gin1_w2.T
    w5 = 3.0 * gin2_w1.T
    w6 = gin2_w2.T
    wd = jnp.tile(dec_w.T, (1, 3))
    bd = jnp.broadcast_to(dec_b, (3,))

    def as_row(v):
        return v.reshape(1, -1).astype(jnp.float32)

    biases = [as_row(enc_b1), as_row(enc_b2), as_row(b3), as_row(gin1_b2),
              as_row(gin2_b1), as_row(gin2_b2), as_row(bd)]

    full = lambda a: pl.BlockSpec(a.shape, lambda i: (0, 0))
    in_specs = [pl.BlockSpec((_BLK, D), lambda i: (i, 0))]
    operands = [x]
    for w, b in zip([w1, w2, w3, w4, w5, w6, wd], biases):
        operands += [w.astype(jnp.float32), b]
        in_specs += [full(w), full(b)]

    out = pl.pallas_call(
        _fused_mlp_kernel,
        grid=(B // _BLK,),
        in_specs=in_specs,
        out_specs=pl.BlockSpec((_BLK, 3), lambda i: (i, 0)),
        out_shape=jax.ShapeDtypeStruct((B, 3), jnp.float32),
    )(*operands)
    return out
